# Initial kernel scaffold; baseline (speedup 1.0000x reference)
import functools

import jax
import jax.numpy as jnp
from jax import lax
from jax.experimental import pallas as pl
from jax.experimental.pallas import tpu as pltpu
from jax.experimental.pallas import tpu_sc as plsc

N = 100000
E = 1600000
NC = 2
NS = 16
NW = NC * NS
CHUNK = 128
NCHUNKS = E // CHUNK
CPW = NCHUNKS // NW
EXTRA = NCHUNKS - CPW * NW
K = 13
SUPER = CPW // K
RPT = N // NS

FLROWS = (N * 4) // 128
FLBLK = 125
MMBLK = 4000
GRID = N // MMBLK

_mesh = plsc.VectorSubcoreMesh(core_axis_name="c", subcore_axis_name="s")


def _acc_init(zeros_hbm, acc_sh, s):
    pltpu.sync_copy(zeros_hbm.at[pl.ds(s * RPT, RPT)], acc_sh.at[pl.ds(s * RPT, RPT)])


def _acc_copyout(acc_sh, out_hbm, c, s):
    pltpu.sync_copy(acc_sh.at[pl.ds(s * RPT, RPT)],
                    out_hbm.at[pl.ds(c * N + s * RPT, RPT)])


@functools.partial(
    pl.kernel,
    out_type=jax.ShapeDtypeStruct((2 * N, 4), jnp.float32),
    mesh=_mesh,
    scratch_types=[
        pltpu.VMEM((K, CHUNK), jnp.int32),
        pltpu.VMEM((CHUNK, 4), jnp.float32),
        pltpu.VMEM_SHARED((N, 4), jnp.float32),
        pltpu.SemaphoreType.DMA,
    ],
)
def _sc_deg(dst_hbm, zeros_hbm, ones_hbm, out_hbm, didx, ones_v, acc_sh, ssem):
    c = lax.axis_index("c")
    s = lax.axis_index("s")
    wid = s * NC + c
    _acc_init(zeros_hbm, acc_sh, s)
    pltpu.sync_copy(ones_hbm, ones_v)
    plsc.subcore_barrier()

    @pl.when(wid < EXTRA)
    def _():
        row = NW * CPW + wid
        pltpu.sync_copy(dst_hbm.at[row], didx.at[0])
        pltpu.async_copy(ones_v, acc_sh.at[didx.at[0]], ssem, add=True).wait()

    @pl.loop(0, SUPER)
    def _(g):
        row0 = wid * CPW + g * K
        pltpu.sync_copy(dst_hbm.at[pl.ds(row0, K)], didx)
        descs = []
        for j in range(K):
            descs.append(
                pltpu.async_copy(ones_v, acc_sh.at[didx.at[j]], ssem, add=True))
        for d in descs:
            d.wait()

    plsc.subcore_barrier()
    _acc_copyout(acc_sh, out_hbm, c, s)


@functools.partial(
    pl.kernel,
    out_type=jax.ShapeDtypeStruct((2 * N, 4), jnp.float32),
    mesh=_mesh,
    scratch_types=[
        pltpu.VMEM((K, CHUNK), jnp.int32),
        pltpu.VMEM((K, CHUNK), jnp.int32),
        pltpu.VMEM((K, CHUNK, 4), jnp.float32),
        pltpu.VMEM_SHARED((N, 4), jnp.float32),
        pltpu.SemaphoreType.DMA,
        pltpu.SemaphoreType.DMA,
    ],
)
def _sc_agg(src_hbm, dst_hbm, u_hbm, zeros_hbm, out_hbm,
            sidx, didx, rows, acc_sh, gsem, ssem):
    c = lax.axis_index("c")
    s = lax.axis_index("s")
    wid = s * NC + c
    _acc_init(zeros_hbm, acc_sh, s)
    plsc.subcore_barrier()

    @pl.when(wid < EXTRA)
    def _():
        row = NW * CPW + wid
        pltpu.sync_copy(src_hbm.at[row], sidx.at[0])
        pltpu.sync_copy(dst_hbm.at[row], didx.at[0])
        pltpu.async_copy(u_hbm.at[sidx.at[0]], rows.at[0], gsem).wait()
        pltpu.async_copy(rows.at[0], acc_sh.at[didx.at[0]], ssem, add=True).wait()

    @pl.loop(0, SUPER)
    def _(g):
        row0 = wid * CPW + g * K
        pltpu.sync_copy(src_hbm.at[pl.ds(row0, K)], sidx)
        pltpu.sync_copy(dst_hbm.at[pl.ds(row0, K)], didx)
        gds = [pltpu.async_copy(u_hbm.at[sidx.at[j]], rows.at[j], gsem)
               for j in range(K)]
        sds = []
        for j in range(K):
            gds[j].wait()
            sds.append(
                pltpu.async_copy(rows.at[j], acc_sh.at[didx.at[j]], ssem,
                                 add=True))
        for d in sds:
            d.wait()

    plsc.subcore_barrier()
    _acc_copyout(acc_sh, out_hbm, c, s)


def _tck1_body(p0, p1, x4, d4, u1):
    deg = p0[...] + p1[...] + 1.0
    d = lax.rsqrt(deg)
    d4[...] = d
    u1[...] = d * x4[...]


def _tck2_body(p0, p1, u1, d4, w1, b1, w2, u2):
    y1 = d4[...] * (p0[...] + p1[...] + u1[...])
    h = jnp.dot(y1, w1[...], preferred_element_type=jnp.float32) + b1[...]
    h = jnp.maximum(h, 0.0)
    z = jnp.dot(h, w2[...], preferred_element_type=jnp.float32)
    u2[...] = d4[...] * z


def _tck3_body(p0, p1, u2, d4, b2t, dx):
    dx[...] = d4[...] * (p0[...] + p1[...] + u2[...]) + b2t[...]


def _flat_spec():
    return pl.BlockSpec((FLBLK, 128), lambda i: (i, 0))


def _row_spec(ch):
    return pl.BlockSpec((MMBLK, ch), lambda i: (i, 0))


def _full_spec(shape):
    return pl.BlockSpec(shape, lambda i: tuple(0 for _ in shape))


_FL = jax.ShapeDtypeStruct((FLROWS, 128), jnp.float32)

_tck1 = pl.pallas_call(
    _tck1_body, grid=(GRID,),
    in_specs=[_flat_spec(), _flat_spec(), _flat_spec()],
    out_specs=[_flat_spec(), _flat_spec()],
    out_shape=[_FL, _FL],
)

_tck2 = pl.pallas_call(
    _tck2_body, grid=(GRID,),
    in_specs=[_row_spec(4), _row_spec(4), _row_spec(4), _row_spec(4),
              _full_spec((4, 32)), _full_spec((1, 32)), _full_spec((32, 4))],
    out_specs=_row_spec(4),
    out_shape=jax.ShapeDtypeStruct((N, 4), jnp.float32),
)

_tck3 = pl.pallas_call(
    _tck3_body, grid=(GRID,),
    in_specs=[_flat_spec(), _flat_spec(), _flat_spec(), _flat_spec(),
              _full_spec((1, 128))],
    out_specs=_flat_spec(),
    out_shape=_FL,
)


def kernel(x, edge_index, W1, b1, W2, b2):
    ei = edge_index.astype(jnp.int32)
    src2d = ei[0].reshape(NCHUNKS, CHUNK)
    dst2d = ei[1].reshape(NCHUNKS, CHUNK)
    x4 = jnp.pad(x, ((0, 0), (0, 1)))
    W1p = jnp.pad(W1, ((0, 1), (0, 0)))
    W2p = jnp.pad(W2, ((0, 0), (0, 1)))
    b1r = b1.reshape(1, 32)
    b2t = jnp.tile(jnp.pad(b2, (0, 1)), 32).reshape(1, 128)
    zeros4 = jnp.zeros((N, 4), jnp.float32)
    ones4 = jnp.ones((CHUNK, 4), jnp.float32)

    degp = _sc_deg(dst2d, zeros4, ones4)
    dp0 = degp[:N].reshape(FLROWS, 128)
    dp1 = degp[N:].reshape(FLROWS, 128)
    d4f, u1f = _tck1(dp0, dp1, x4.reshape(FLROWS, 128))
    u1 = u1f.reshape(N, 4)

    s1 = _sc_agg(src2d, dst2d, u1, zeros4)
    u2 = _tck2(s1[:N], s1[N:], u1, d4f.reshape(N, 4), W1p, b1r, W2p)

    s2 = _sc_agg(src2d, dst2d, u2, zeros4)
    dxf = _tck3(s2[:N].reshape(FLROWS, 128), s2[N:].reshape(FLROWS, 128),
                u2.reshape(FLROWS, 128), d4f, b2t)
    return dxf.reshape(N, 4)[:, :3]


# speedup vs baseline: 36.2255x; 36.2255x over previous
import functools

import jax
import jax.numpy as jnp
from jax import lax
from jax.experimental import pallas as pl
from jax.experimental.pallas import tpu as pltpu
from jax.experimental.pallas import tpu_sc as plsc

N = 100000
NP = 100096
CH = 8
E = 1600000
NC = 2
NS = 16
NW = NC * NS
CHUNK = 128
CPW = 392
EP = NW * CPW * CHUNK
NCHUNKS = EP // CHUNK
K = 56
SUPER = CPW // K
RPT = NP // NS

FLROWS = (NP * CH) // 128
MMBLK = 6256
MMGRID = NP // MMBLK

_mesh = plsc.VectorSubcoreMesh(core_axis_name="c", subcore_axis_name="s")
_sc_params = pltpu.CompilerParams(use_tc_tiling_on_sc=False)


def _acc_init(zeros_hbm, acc_sh, s):
    pltpu.sync_copy(zeros_hbm.at[pl.ds(s * RPT, RPT)],
                    acc_sh.at[pl.ds(s * RPT, RPT)])


def _acc_copyout(acc_sh, out_hbm, c, s):
    pltpu.sync_copy(acc_sh.at[pl.ds(s * RPT, RPT)],
                    out_hbm.at[pl.ds(c * NP + s * RPT, RPT)])


@functools.partial(
    pl.kernel,
    out_type=jax.ShapeDtypeStruct((2 * NP, CH), jnp.float32),
    mesh=_mesh,
    scratch_types=[
        pltpu.VMEM((K, CHUNK), jnp.int32),
        pltpu.VMEM((CHUNK, CH), jnp.float32),
        pltpu.VMEM_SHARED((NP, CH), jnp.float32),
        pltpu.SemaphoreType.DMA,
    ],
    compiler_params=_sc_params,
)
def _sc_deg(dst_hbm, zeros_hbm, ones_hbm, out_hbm, didx, ones_v, acc_sh, ssem):
    c = lax.axis_index("c")
    s = lax.axis_index("s")
    wid = s * NC + c
    _acc_init(zeros_hbm, acc_sh, s)
    pltpu.sync_copy(ones_hbm, ones_v)
    plsc.subcore_barrier()

    @pl.loop(0, SUPER)
    def _(g):
        row0 = wid * CPW + g * K
        pltpu.sync_copy(dst_hbm.at[pl.ds(row0, K)], didx)
        descs = [
            pltpu.async_copy(ones_v, acc_sh.at[didx.at[j]], ssem, add=True)
            for j in range(K)
        ]
        for d in descs:
            d.wait()

    plsc.subcore_barrier()
    _acc_copyout(acc_sh, out_hbm, c, s)


@functools.partial(
    pl.kernel,
    out_type=jax.ShapeDtypeStruct((2 * NP, CH), jnp.float32),
    mesh=_mesh,
    scratch_types=[
        pltpu.VMEM((K, CHUNK), jnp.int32),
        pltpu.VMEM((K, CHUNK), jnp.int32),
        pltpu.VMEM((K, CHUNK, CH), jnp.float32),
        pltpu.VMEM_SHARED((NP, CH), jnp.float32),
        pltpu.SemaphoreType.DMA,
        pltpu.SemaphoreType.DMA,
    ],
    compiler_params=_sc_params,
)
def _sc_agg(src_hbm, dst_hbm, u_hbm, zeros_hbm, out_hbm,
            sidx, didx, rows, acc_sh, gsem, ssem):
    c = lax.axis_index("c")
    s = lax.axis_index("s")
    wid = s * NC + c
    _acc_init(zeros_hbm, acc_sh, s)
    plsc.subcore_barrier()

    @pl.loop(0, SUPER)
    def _(g):
        row0 = wid * CPW + g * K
        pltpu.sync_copy(src_hbm.at[pl.ds(row0, K)], sidx)
        pltpu.sync_copy(dst_hbm.at[pl.ds(row0, K)], didx)
        gds = [pltpu.async_copy(u_hbm.at[sidx.at[j]], rows.at[j], gsem)
               for j in range(K)]
        sds = []
        for j in range(K):
            gds[j].wait()
            sds.append(
                pltpu.async_copy(rows.at[j], acc_sh.at[didx.at[j]], ssem,
                                 add=True))
        for d in sds:
            d.wait()

    plsc.subcore_barrier()
    _acc_copyout(acc_sh, out_hbm, c, s)


def _tck1_body(p0, p1, x8, d8, u1):
    deg = p0[...] + p1[...] + 1.0
    d = lax.rsqrt(deg)
    d8[...] = d
    u1[...] = d * x8[...]


def _tck2_body(p0, p1, u1, d8, w1, b1, w2, u2):
    y1 = d8[...] * (p0[...] + p1[...] + u1[...])
    h = jnp.dot(y1, w1[...], preferred_element_type=jnp.float32) + b1[...]
    h = jnp.maximum(h, 0.0)
    z = jnp.dot(h, w2[...], preferred_element_type=jnp.float32)
    u2[...] = d8[...] * z


def _tck3_body(p0, p1, u2, d8, b2t, dx):
    dx[...] = d8[...] * (p0[...] + p1[...] + u2[...]) + b2t[...]


def _flat_spec():
    return pl.BlockSpec((FLROWS, 128), lambda i: (0, 0))


def _row_spec(ch):
    return pl.BlockSpec((MMBLK, ch), lambda i: (i, 0))


def _full_spec(shape):
    return pl.BlockSpec(shape, lambda i: tuple(0 for _ in shape))


_FL = jax.ShapeDtypeStruct((FLROWS, 128), jnp.float32)

_tck1 = pl.pallas_call(
    _tck1_body, grid=(1,),
    in_specs=[_flat_spec(), _flat_spec(), _flat_spec()],
    out_specs=[_flat_spec(), _flat_spec()],
    out_shape=[_FL, _FL],
)

_tck2 = pl.pallas_call(
    _tck2_body, grid=(MMGRID,),
    in_specs=[_row_spec(CH), _row_spec(CH), _row_spec(CH), _row_spec(CH),
              _full_spec((CH, 32)), _full_spec((1, 32)), _full_spec((32, CH))],
    out_specs=_row_spec(CH),
    out_shape=jax.ShapeDtypeStruct((NP, CH), jnp.float32),
)

_tck3 = pl.pallas_call(
    _tck3_body, grid=(1,),
    in_specs=[_flat_spec(), _flat_spec(), _flat_spec(), _flat_spec(),
              _full_spec((1, 128))],
    out_specs=_flat_spec(),
    out_shape=_FL,
)


def kernel(x, edge_index, W1, b1, W2, b2):
    ei = edge_index.astype(jnp.int32)
    pad_idx = jnp.full((EP - E,), N, jnp.int32)
    src2d = jnp.concatenate([ei[0], pad_idx]).reshape(NCHUNKS, CHUNK)
    dst2d = jnp.concatenate([ei[1], pad_idx]).reshape(NCHUNKS, CHUNK)
    x8 = jnp.pad(x, ((0, NP - N), (0, CH - 3)))
    W1p = jnp.pad(W1, ((0, CH - 3), (0, 0)))
    W2p = jnp.pad(W2, ((0, 0), (0, CH - 3)))
    b1r = b1.reshape(1, 32)
    b2t = jnp.tile(jnp.pad(b2, (0, CH - 3)), 128 // CH).reshape(1, 128)
    zeros8 = jnp.zeros((NP, CH), jnp.float32)
    ones8 = jnp.ones((CHUNK, CH), jnp.float32)

    degp = _sc_deg(dst2d, zeros8, ones8)
    dp0 = degp[:NP].reshape(FLROWS, 128)
    dp1 = degp[NP:].reshape(FLROWS, 128)
    d8f, u1f = _tck1(dp0, dp1, x8.reshape(FLROWS, 128))
    u1 = u1f.reshape(NP, CH)

    s1 = _sc_agg(src2d, dst2d, u1, zeros8)
    u2 = _tck2(s1[:NP], s1[NP:], u1, d8f.reshape(NP, CH), W1p, b1r, W2p)

    s2 = _sc_agg(src2d, dst2d, u2, zeros8)
    dxf = _tck3(s2[:NP].reshape(FLROWS, 128), s2[NP:].reshape(FLROWS, 128),
                u2.reshape(FLROWS, 128), d8f, b2t)
    return dxf.reshape(NP, CH)[:N, :3]


# speedup vs baseline: 49.5387x; 1.3675x over previous
import functools

import jax
import jax.numpy as jnp
from jax import lax
from jax.experimental import pallas as pl
from jax.experimental.pallas import tpu as pltpu
from jax.experimental.pallas import tpu_sc as plsc

N = 100000
NP = 100096
CH = 8
E = 1600000
NC = 2
NS = 16
NW = NC * NS
CHUNK = 128
NCHUNKS = E // CHUNK
CPW = NCHUNKS // NW
EXTRA = NCHUNKS - CPW * NW
K = 26
SUPER = CPW // K
RPT = NP // NS

MMBLK = 6256
MMGRID = NP // MMBLK

_mesh = plsc.VectorSubcoreMesh(core_axis_name="c", subcore_axis_name="s")
_sc_params = pltpu.CompilerParams(use_tc_tiling_on_sc=False)
_P = jax.ShapeDtypeStruct((NP, CH), jnp.float32)


def _acc_init(zeros_hbm, acc_sh, s):
    pltpu.sync_copy(zeros_hbm.at[pl.ds(s * RPT, RPT)],
                    acc_sh.at[pl.ds(s * RPT, RPT)])


def _acc_copyout(acc_sh, out0_hbm, out1_hbm, c, s):
    @pl.when(c == 0)
    def _():
        pltpu.sync_copy(acc_sh.at[pl.ds(s * RPT, RPT)],
                        out0_hbm.at[pl.ds(s * RPT, RPT)])

    @pl.when(c == 1)
    def _():
        pltpu.sync_copy(acc_sh.at[pl.ds(s * RPT, RPT)],
                        out1_hbm.at[pl.ds(s * RPT, RPT)])


@functools.partial(
    pl.kernel,
    out_type=[_P, _P],
    mesh=_mesh,
    scratch_types=[
        pltpu.VMEM((K, CHUNK), jnp.int32),
        pltpu.VMEM((CHUNK, CH), jnp.float32),
        pltpu.VMEM_SHARED((NP, CH), jnp.float32),
        pltpu.SemaphoreType.DMA,
    ],
    compiler_params=_sc_params,
)
def _sc_deg(dst_hbm, zeros_hbm, ones_hbm, out0_hbm, out1_hbm,
            didx, ones_v, acc_sh, ssem):
    c = lax.axis_index("c")
    s = lax.axis_index("s")
    wid = s * NC + c
    _acc_init(zeros_hbm, acc_sh, s)
    pltpu.sync_copy(ones_hbm, ones_v)
    plsc.subcore_barrier()

    @pl.when(wid < EXTRA)
    def _():
        row = NW * CPW + wid
        pltpu.sync_copy(dst_hbm.at[row], didx.at[0])
        pltpu.async_copy(ones_v, acc_sh.at[didx.at[0]], ssem, add=True).wait()

    @pl.loop(0, SUPER)
    def _(g):
        row0 = wid * CPW + g * K
        pltpu.sync_copy(dst_hbm.at[pl.ds(row0, K)], didx)
        descs = [
            pltpu.async_copy(ones_v, acc_sh.at[didx.at[j]], ssem, add=True)
            for j in range(K)
        ]
        for d in descs:
            d.wait()

    plsc.subcore_barrier()
    _acc_copyout(acc_sh, out0_hbm, out1_hbm, c, s)


@functools.partial(
    pl.kernel,
    out_type=[_P, _P],
    mesh=_mesh,
    scratch_types=[
        pltpu.VMEM((K, CHUNK), jnp.int32),
        pltpu.VMEM((K, CHUNK), jnp.int32),
        pltpu.VMEM((K, CHUNK, CH), jnp.float32),
        pltpu.VMEM_SHARED((NP, CH), jnp.float32),
        pltpu.SemaphoreType.DMA,
        pltpu.SemaphoreType.DMA,
    ],
    compiler_params=_sc_params,
)
def _sc_agg(src_hbm, dst_hbm, u_hbm, zeros_hbm, out0_hbm, out1_hbm,
            sidx, didx, rows, acc_sh, gsem, ssem):
    c = lax.axis_index("c")
    s = lax.axis_index("s")
    wid = s * NC + c
    _acc_init(zeros_hbm, acc_sh, s)
    plsc.subcore_barrier()

    @pl.when(wid < EXTRA)
    def _():
        row = NW * CPW + wid
        pltpu.sync_copy(src_hbm.at[row], sidx.at[0])
        pltpu.sync_copy(dst_hbm.at[row], didx.at[0])
        pltpu.async_copy(u_hbm.at[sidx.at[0]], rows.at[0], gsem).wait()
        pltpu.async_copy(rows.at[0], acc_sh.at[didx.at[0]], ssem,
                         add=True).wait()

    @pl.loop(0, SUPER)
    def _(g):
        row0 = wid * CPW + g * K
        pltpu.sync_copy(src_hbm.at[pl.ds(row0, K)], sidx)
        pltpu.sync_copy(dst_hbm.at[pl.ds(row0, K)], didx)
        gds = [pltpu.async_copy(u_hbm.at[sidx.at[j]], rows.at[j], gsem)
               for j in range(K)]
        sds = []
        for j in range(K):
            gds[j].wait()
            sds.append(
                pltpu.async_copy(rows.at[j], acc_sh.at[didx.at[j]], ssem,
                                 add=True))
        for d in sds:
            d.wait()

    plsc.subcore_barrier()
    _acc_copyout(acc_sh, out0_hbm, out1_hbm, c, s)


def _tck1_body(p0, p1, x, d8, u1):
    deg = p0[...] + p1[...] + 1.0
    d = lax.rsqrt(deg)
    d8[...] = d
    xp = jnp.pad(x[...], ((0, 0), (0, CH - 3)))
    u1[...] = d * xp


def _tck2_body(p0, p1, u1, d8, w1, b1, w2, u2):
    y1 = d8[...] * (p0[...] + p1[...] + u1[...])
    h = jnp.dot(y1, w1[...], preferred_element_type=jnp.float32) + b1[...]
    h = jnp.maximum(h, 0.0)
    z = jnp.dot(h, w2[...], preferred_element_type=jnp.float32)
    u2[...] = d8[...] * z


def _tck3_body(p0, p1, u2, d8, b2r, dx):
    v = d8[...] * (p0[...] + p1[...] + u2[...]) + b2r[...]
    dx[...] = v[:, :3]


def _row_spec(ch):
    return pl.BlockSpec((MMBLK, ch), lambda i: (i, 0))


def _full_spec(shape):
    return pl.BlockSpec(shape, lambda i: tuple(0 for _ in shape))


_tck1 = pl.pallas_call(
    _tck1_body, grid=(MMGRID,),
    in_specs=[_row_spec(CH), _row_spec(CH), _row_spec(3)],
    out_specs=[_row_spec(CH), _row_spec(CH)],
    out_shape=[_P, _P],
)

_tck2 = pl.pallas_call(
    _tck2_body, grid=(MMGRID,),
    in_specs=[_row_spec(CH), _row_spec(CH), _row_spec(CH), _row_spec(CH),
              _full_spec((CH, 32)), _full_spec((1, 32)), _full_spec((32, CH))],
    out_specs=_row_spec(CH),
    out_shape=_P,
)

_tck3 = pl.pallas_call(
    _tck3_body, grid=(MMGRID,),
    in_specs=[_row_spec(CH), _row_spec(CH), _row_spec(CH), _row_spec(CH),
              _full_spec((1, CH))],
    out_specs=_row_spec(3),
    out_shape=jax.ShapeDtypeStruct((N, 3), jnp.float32),
)


def kernel(x, edge_index, W1, b1, W2, b2):
    ei = edge_index.astype(jnp.int32)
    src2d = ei[0].reshape(NCHUNKS, CHUNK)
    dst2d = ei[1].reshape(NCHUNKS, CHUNK)
    W1p = jnp.pad(W1, ((0, CH - 3), (0, 0)))
    W2p = jnp.pad(W2, ((0, 0), (0, CH - 3)))
    b1r = b1.reshape(1, 32)
    b2r = jnp.pad(b2, (0, CH - 3)).reshape(1, CH)
    zeros8 = jnp.zeros((NP, CH), jnp.float32)
    ones8 = jnp.ones((CHUNK, CH), jnp.float32)

    dp0, dp1 = _sc_deg(dst2d, zeros8, ones8)
    d8, u1 = _tck1(dp0, dp1, x)

    s1a, s1b = _sc_agg(src2d, dst2d, u1, zeros8)
    u2 = _tck2(s1a, s1b, u1, d8, W1p, b1r, W2p)

    s2a, s2b = _sc_agg(src2d, dst2d, u2, zeros8)
    return _tck3(s2a, s2b, u2, d8, b2r)


# speedup vs baseline: 66.3867x; 1.3401x over previous
import functools

import jax
import jax.numpy as jnp
from jax import lax
from jax.experimental import pallas as pl
from jax.experimental.pallas import tpu as pltpu
from jax.experimental.pallas import tpu_sc as plsc

N = 100000
NP = 100096
CH = 8
E = 1600000
NC = 2
NS = 16
NW = NC * NS
CHUNK = 128
NCHUNKS = E // CHUNK
CPW = NCHUNKS // NW
EXTRA = NCHUNKS - CPW * NW
K = 26
SUPER = CPW // K
RPT = NP // NS
DR = NP // NW
B0, B1A, B1B = 1568, 1560, 1464

_mesh = plsc.VectorSubcoreMesh(core_axis_name="c", subcore_axis_name="s")
_agg_params = pltpu.CompilerParams(use_tc_tiling_on_sc=False)
_cmp_params = pltpu.CompilerParams(use_tc_tiling_on_sc=False,
                                   needs_layout_passes=False)
_P = jax.ShapeDtypeStruct((NP, CH), jnp.float32)


def _newton_rsqrt(v):
    bi = plsc.bitcast(v, jnp.int32)
    y = plsc.bitcast(jnp.int32(0x5F3759DF) - lax.shift_right_logical(bi, 1),
                     jnp.float32)
    vh = 0.5 * v
    for _ in range(3):
        y = y * (1.5 - vh * y * y)
    return y


def _flatpos(i):
    iota = lax.iota(jnp.int32, 16)
    return lax.shift_right_logical(iota, 3) + 2 * i, iota & 7


@functools.partial(
    pl.kernel,
    out_type=[_P, _P],
    mesh=_mesh,
    scratch_types=[
        pltpu.VMEM((K, CHUNK), jnp.int32),
        pltpu.VMEM((CHUNK, CH), jnp.float32),
        pltpu.VMEM_SHARED((NP, CH), jnp.float32),
        pltpu.SemaphoreType.DMA,
    ],
    compiler_params=_agg_params,
)
def _sc_deg(ei3, zeros_hbm, ones_hbm, out0, out1, didx, ones_v, acc_sh, ssem):
    c = lax.axis_index("c")
    s = lax.axis_index("s")
    wid = s * NC + c
    pltpu.sync_copy(zeros_hbm.at[pl.ds(s * RPT, RPT)],
                    acc_sh.at[pl.ds(s * RPT, RPT)])
    pltpu.sync_copy(ones_hbm, ones_v)
    plsc.subcore_barrier()

    @pl.when(wid < EXTRA)
    def _():
        row = NW * CPW + wid
        pltpu.sync_copy(ei3.at[1, row], didx.at[0])
        pltpu.async_copy(ones_v, acc_sh.at[didx.at[0]], ssem, add=True).wait()

    @pl.loop(0, SUPER)
    def _(g):
        row0 = wid * CPW + g * K
        pltpu.sync_copy(ei3.at[1, pl.ds(row0, K)], didx)
        descs = [
            pltpu.async_copy(ones_v, acc_sh.at[didx.at[j]], ssem, add=True)
            for j in range(K)
        ]
        for dsc in descs:
            dsc.wait()

    plsc.subcore_barrier()

    @pl.when(c == 0)
    def _():
        pltpu.sync_copy(acc_sh.at[pl.ds(s * RPT, RPT)],
                        out0.at[pl.ds(s * RPT, RPT)])

    @pl.when(c == 1)
    def _():
        pltpu.sync_copy(acc_sh.at[pl.ds(s * RPT, RPT)],
                        out1.at[pl.ds(s * RPT, RPT)])


@functools.partial(
    pl.kernel,
    out_type=[_P, _P],
    mesh=_mesh,
    scratch_types=[
        pltpu.VMEM((K, CHUNK), jnp.int32),
        pltpu.VMEM((K, CHUNK), jnp.int32),
        pltpu.VMEM((K, CHUNK, CH), jnp.float32),
        pltpu.VMEM_SHARED((NP, CH), jnp.float32),
        pltpu.SemaphoreType.DMA,
        pltpu.SemaphoreType.DMA,
    ],
    compiler_params=_agg_params,
)
def _sc_agg(ei3, u_hbm, zeros_hbm, outa, outb,
            sidx, didx, rows, acc_sh, gsem, ssem):
    c = lax.axis_index("c")
    s = lax.axis_index("s")
    wid = s * NC + c
    pltpu.sync_copy(zeros_hbm.at[pl.ds(s * RPT, RPT)],
                    acc_sh.at[pl.ds(s * RPT, RPT)])
    plsc.subcore_barrier()

    @pl.when(wid < EXTRA)
    def _():
        row = NW * CPW + wid
        pltpu.sync_copy(ei3.at[0, row], sidx.at[0])
        pltpu.sync_copy(ei3.at[1, row], didx.at[0])
        pltpu.async_copy(u_hbm.at[sidx.at[0]], rows.at[0], gsem).wait()
        pltpu.async_copy(rows.at[0], acc_sh.at[didx.at[0]], ssem,
                         add=True).wait()

    @pl.loop(0, SUPER)
    def _(g):
        row0 = wid * CPW + g * K
        pltpu.sync_copy(ei3.at[0, pl.ds(row0, K)], sidx)
        pltpu.sync_copy(ei3.at[1, pl.ds(row0, K)], didx)
        gds = [pltpu.async_copy(u_hbm.at[sidx.at[j]], rows.at[j], gsem)
               for j in range(K)]
        sds = []
        for j in range(K):
            gds[j].wait()
            sds.append(
                pltpu.async_copy(rows.at[j], acc_sh.at[didx.at[j]], ssem,
                                 add=True))
        for dsc in sds:
            dsc.wait()

    plsc.subcore_barrier()

    @pl.when(c == 0)
    def _():
        pltpu.sync_copy(acc_sh.at[pl.ds(s * RPT, RPT)],
                        outa.at[pl.ds(s * RPT, RPT)])

    @pl.when(c == 1)
    def _():
        pltpu.sync_copy(acc_sh.at[pl.ds(s * RPT, RPT)],
                        outb.at[pl.ds(s * RPT, RPT)])


@functools.partial(
    pl.kernel,
    out_type=[_P, _P],
    mesh=_mesh,
    scratch_types=[
        pltpu.VMEM((B0, CH), jnp.float32),
        pltpu.VMEM((B0, CH), jnp.float32),
        pltpu.VMEM((B0, 3), jnp.float32),
        pltpu.VMEM((B0, CH), jnp.float32),
        pltpu.VMEM((B0, CH), jnp.float32),
    ],
    compiler_params=_cmp_params,
)
def _sc_prep(dp0, dp1, x_hbm, u1t, d8t, pa, pb, px, pu, pd):
    c = lax.axis_index("c")
    s = lax.axis_index("s")
    wid = s * NC + c

    def block(off, size, xsize):
        base = wid * DR + off
        pltpu.sync_copy(dp0.at[pl.ds(base, size)], pa.at[pl.ds(0, size)])
        pltpu.sync_copy(dp1.at[pl.ds(base, size)], pb.at[pl.ds(0, size)])
        pltpu.sync_copy(x_hbm.at[pl.ds(base, xsize)], px.at[pl.ds(0, xsize)])

        @pl.loop(0, size // 2)
        def _(i):
            rowv, colv = _flatpos(i)
            a = plsc.load_gather(pa, [rowv, colv])
            b = plsc.load_gather(pb, [rowv, colv])
            dv = _newton_rsqrt(a + b + 1.0)
            xg = plsc.load_gather(px, [rowv, jnp.minimum(colv, 2)])
            plsc.store_scatter(pd, [rowv, colv], dv)
            plsc.store_scatter(pu, [rowv, colv], dv * xg)

        pltpu.sync_copy(pu.at[pl.ds(0, size)], u1t.at[pl.ds(base, size)])
        pltpu.sync_copy(pd.at[pl.ds(0, size)], d8t.at[pl.ds(base, size)])

    block(0, B0, B0)

    @pl.when(wid == NW - 1)
    def _():
        block(B0, B1A, B1B)

    @pl.when(wid < NW - 1)
    def _():
        block(B0, B1A, B1A)


DS = 1564


@functools.partial(
    pl.kernel,
    out_type=_P,
    mesh=_mesh,
    scratch_types=[
        pltpu.VMEM((DS, CH), jnp.float32),
        pltpu.VMEM((DS, CH), jnp.float32),
        pltpu.VMEM((DS, CH), jnp.float32),
        pltpu.VMEM((DS, CH), jnp.float32),
        pltpu.VMEM((DS, CH), jnp.float32),
        pltpu.VMEM((DS * CH,), jnp.float32),
        pltpu.VMEM((DS * CH,), jnp.float32),
        pltpu.VMEM((64,), jnp.float32),
        pltpu.VMEM((CH, 32), jnp.float32),
        pltpu.VMEM((32,), jnp.float32),
        pltpu.VMEM((3, 32), jnp.float32),
        pltpu.SMEM((CH, 32), jnp.float32),
        pltpu.SMEM((32,), jnp.float32),
        pltpu.SMEM((3, 32), jnp.float32),
    ],
    compiler_params=_cmp_params,
)
def _sc_dense(d8t, s1a, s1b, u1t, w1_hbm, b1_hbm, w2t_hbm, u2t,
              qd, qs, qt, qu, qo, ybuf, dbuf, zbuf, w1v, b1vv, w2v,
              w1, b1v, w2):
    c = lax.axis_index("c")
    s = lax.axis_index("s")
    wid = s * NC + c
    pltpu.sync_copy(w1_hbm, w1v)
    pltpu.sync_copy(b1_hbm, b1vv)
    pltpu.sync_copy(w2t_hbm, w2v)
    for r in range(CH):
        for cb in (0, 16):
            vv = w1v[r, pl.ds(cb, 16)]
            for j in range(16):
                w1[r, cb + j] = vv[j]
    for cb in (0, 16):
        vv = b1vv[pl.ds(cb, 16)]
        for j in range(16):
            b1v[cb + j] = vv[j]
    for r in range(3):
        for cb in (0, 16):
            vv = w2v[r, pl.ds(cb, 16)]
            for j in range(16):
                w2[r, cb + j] = vv[j]
    iota = lax.iota(jnp.int32, 16)
    zero16 = jnp.zeros((16,), jnp.float32)
    for t in range(4):
        zbuf[pl.ds(t * 16, 16)] = zero16
    colv8 = iota & 7
    rbase = lax.shift_right_logical(iota, 3)
    basepat = jnp.where(colv8 < 3, colv8 * 16 + rbase, 48)

    for off in (0, DS):
        base = wid * DR + off
        pltpu.sync_copy(d8t.at[pl.ds(base, DS)], qd)
        pltpu.sync_copy(s1a.at[pl.ds(base, DS)], qs)
        pltpu.sync_copy(s1b.at[pl.ds(base, DS)], qt)
        pltpu.sync_copy(u1t.at[pl.ds(base, DS)], qu)

        @pl.loop(0, DS // 2)
        def _(i):
            rowv, colv = _flatpos(i)
            dv = plsc.load_gather(qd, [rowv, colv])
            sv = (plsc.load_gather(qs, [rowv, colv])
                  + plsc.load_gather(qt, [rowv, colv])
                  + plsc.load_gather(qu, [rowv, colv]))
            dbuf[pl.ds(i * 16, 16)] = dv
            ybuf[pl.ds(i * 16, 16)] = dv * sv

        NGROUPS = DS // 16 + 1

        @pl.loop(0, NGROUPS)
        def _(g):
            ng = jnp.minimum(g * 16, DS - 16)
            fb = ng * CH
            yc = [plsc.load_gather(ybuf, [iota * CH + (fb + cc)])
                  for cc in range(CH)]
            z0 = zero16
            z1 = zero16
            z2 = zero16
            for k in range(32):
                hk = yc[0] * w1[0, k]
                for cc in range(1, CH):
                    hk = hk + yc[cc] * w1[cc, k]
                hk = jnp.maximum(hk + b1v[k], 0.0)
                z0 = z0 + hk * w2[0, k]
                z1 = z1 + hk * w2[1, k]
                z2 = z2 + hk * w2[2, k]
            zbuf[pl.ds(0, 16)] = z0
            zbuf[pl.ds(16, 16)] = z1
            zbuf[pl.ds(32, 16)] = z2
            for v in range(CH):
                zg = plsc.load_gather(zbuf, [basepat + 2 * v])
                dv = dbuf[pl.ds(fb + v * 16, 16)]
                rowv = rbase + (ng + 2 * v)
                plsc.store_scatter(qo, [rowv, colv8], dv * zg)

        pltpu.sync_copy(qo, u2t.at[pl.ds(base, DS)])


@functools.partial(
    pl.kernel,
    out_type=jax.ShapeDtypeStruct((N * 3,), jnp.float32),
    mesh=_mesh,
    scratch_types=[
        pltpu.VMEM((B0, CH), jnp.float32),
        pltpu.VMEM((B0, CH), jnp.float32),
        pltpu.VMEM((B0, CH), jnp.float32),
        pltpu.VMEM((B0, CH), jnp.float32),
        pltpu.VMEM((B0 * CH,), jnp.float32),
        pltpu.VMEM((B0 * 3,), jnp.float32),
        pltpu.VMEM((16,), jnp.float32),
    ],
    compiler_params=_cmp_params,
)
def _sc_final(d8t, s2a, s2b, u2t, b2_hbm, dxout,
              rd, rs, rt, ru, vbuf, dxbuf, b2b):
    c = lax.axis_index("c")
    s = lax.axis_index("s")
    wid = s * NC + c
    pltpu.sync_copy(b2_hbm, b2b)
    iota = lax.iota(jnp.int32, 16)
    colv8 = iota & 7
    b2vec = plsc.load_gather(b2b, [jnp.where(colv8 < 3, colv8, 3)])
    pats = []
    for r in range(3):
        f = iota + 16 * r
        n3 = f // 3
        pats.append(CH * n3 + (f - 3 * n3))

    def block(off, size):
        base = wid * DR + off
        pltpu.sync_copy(d8t.at[pl.ds(base, size)], rd.at[pl.ds(0, size)])
        pltpu.sync_copy(s2a.at[pl.ds(base, size)], rs.at[pl.ds(0, size)])
        pltpu.sync_copy(s2b.at[pl.ds(base, size)], rt.at[pl.ds(0, size)])
        pltpu.sync_copy(u2t.at[pl.ds(base, size)], ru.at[pl.ds(0, size)])

        @pl.loop(0, size // 2)
        def _(i):
            rowv, colv = _flatpos(i)
            dv = plsc.load_gather(rd, [rowv, colv])
            sv = (plsc.load_gather(rs, [rowv, colv])
                  + plsc.load_gather(rt, [rowv, colv])
                  + plsc.load_gather(ru, [rowv, colv]))
            vbuf[pl.ds(i * 16, 16)] = dv * sv + b2vec

        @pl.loop(0, size // 16 + 1)
        def _(q):
            mq = jnp.minimum(q * 16, size - 16)
            for r in range(3):
                dxv = plsc.load_gather(vbuf, [pats[r] + CH * mq])
                dxbuf[pl.ds(3 * mq + 16 * r, 16)] = dxv

        pltpu.sync_copy(dxbuf.at[pl.ds(0, size * 3)],
                        dxout.at[pl.ds(base * 3, size * 3)])

    block(0, B0)

    @pl.when(wid == NW - 1)
    def _():
        block(B0, B1B)

    @pl.when(wid < NW - 1)
    def _():
        block(B0, B1A)


def kernel(x, edge_index, W1, b1, W2, b2):
    ei3 = edge_index.astype(jnp.int32).reshape(2, NCHUNKS, CHUNK)
    W1p = jnp.pad(W1, ((0, CH - 3), (0, 0)))
    W2T = W2.T
    b2p = jnp.pad(b2, (0, 13))
    zeros8 = jnp.zeros((NP, CH), jnp.float32)
    ones8 = jnp.ones((CHUNK, CH), jnp.float32)

    dp0, dp1 = _sc_deg(ei3, zeros8, ones8)
    u1t, d8t = _sc_prep(dp0, dp1, x)
    s1a, s1b = _sc_agg(ei3, u1t, zeros8)
    u2t = _sc_dense(d8t, s1a, s1b, u1t, W1p, b1, W2T)
    s2a, s2b = _sc_agg(ei3, u2t, zeros8)
    dxf = _sc_final(d8t, s2a, s2b, u2t, b2p)
    return dxf.reshape(N, 3)


# speedup vs baseline: 74.2715x; 1.1188x over previous
import functools

import jax
import jax.numpy as jnp
from jax import lax
from jax.experimental import pallas as pl
from jax.experimental.pallas import tpu as pltpu
from jax.experimental.pallas import tpu_sc as plsc

N = 100000
NP = 100096
CH = 8
E = 1600000
NC = 2
NS = 16
NW = NC * NS
CHUNK = 128
NCHUNKS = E // CHUNK
CPW = NCHUNKS // NW
EXTRA = NCHUNKS - CPW * NW
K = 26
SUPER = CPW // K
RPT = NP // NS
DR = NP // NW
B0, B1A, B1B = 1568, 1560, 1464

_mesh = plsc.VectorSubcoreMesh(core_axis_name="c", subcore_axis_name="s")
_agg_params = pltpu.CompilerParams(use_tc_tiling_on_sc=False)
_cmp_params = pltpu.CompilerParams(use_tc_tiling_on_sc=False,
                                   needs_layout_passes=False)
_P = jax.ShapeDtypeStruct((NP, CH), jnp.float32)


def _newton_rsqrt(v):
    bi = plsc.bitcast(v, jnp.int32)
    y = plsc.bitcast(jnp.int32(0x5F3759DF) - lax.shift_right_logical(bi, 1),
                     jnp.float32)
    vh = 0.5 * v
    for _ in range(3):
        y = y * (1.5 - vh * y * y)
    return y


def _flatpos(i):
    iota = lax.iota(jnp.int32, 16)
    return lax.shift_right_logical(iota, 3) + 2 * i, iota & 7


@functools.partial(
    pl.kernel,
    out_type=[_P, _P],
    mesh=_mesh,
    scratch_types=[
        pltpu.VMEM((K, CHUNK), jnp.int32),
        pltpu.VMEM((CHUNK, CH), jnp.float32),
        pltpu.VMEM_SHARED((NP, CH), jnp.float32),
        pltpu.SemaphoreType.DMA,
    ],
    compiler_params=_agg_params,
)
def _sc_deg(ei3, zeros_hbm, ones_hbm, out0, out1, didx, ones_v, acc_sh, ssem):
    c = lax.axis_index("c")
    s = lax.axis_index("s")
    wid = s * NC + c
    pltpu.sync_copy(zeros_hbm.at[pl.ds(s * RPT, RPT)],
                    acc_sh.at[pl.ds(s * RPT, RPT)])
    pltpu.sync_copy(ones_hbm, ones_v)
    plsc.subcore_barrier()

    @pl.when(wid < EXTRA)
    def _():
        row = NW * CPW + wid
        pltpu.sync_copy(ei3.at[1, row], didx.at[0])
        pltpu.async_copy(ones_v, acc_sh.at[didx.at[0]], ssem, add=True).wait()

    @pl.loop(0, SUPER)
    def _(g):
        row0 = wid * CPW + g * K
        pltpu.sync_copy(ei3.at[1, pl.ds(row0, K)], didx)
        descs = [
            pltpu.async_copy(ones_v, acc_sh.at[didx.at[j]], ssem, add=True)
            for j in range(K)
        ]
        for dsc in descs:
            dsc.wait()

    plsc.subcore_barrier()

    @pl.when(c == 0)
    def _():
        pltpu.sync_copy(acc_sh.at[pl.ds(s * RPT, RPT)],
                        out0.at[pl.ds(s * RPT, RPT)])

    @pl.when(c == 1)
    def _():
        pltpu.sync_copy(acc_sh.at[pl.ds(s * RPT, RPT)],
                        out1.at[pl.ds(s * RPT, RPT)])


@functools.partial(
    pl.kernel,
    out_type=[_P, _P],
    mesh=_mesh,
    scratch_types=[
        pltpu.VMEM((K, CHUNK), jnp.int32),
        pltpu.VMEM((K, CHUNK), jnp.int32),
        pltpu.VMEM((K, CHUNK, CH), jnp.float32),
        pltpu.VMEM_SHARED((NP, CH), jnp.float32),
        pltpu.SemaphoreType.DMA,
        pltpu.SemaphoreType.DMA,
    ],
    compiler_params=_agg_params,
)
def _sc_agg(ei3, u_hbm, zeros_hbm, outa, outb,
            sidx, didx, rows, acc_sh, gsem, ssem):
    c = lax.axis_index("c")
    s = lax.axis_index("s")
    wid = s * NC + c
    pltpu.sync_copy(zeros_hbm.at[pl.ds(s * RPT, RPT)],
                    acc_sh.at[pl.ds(s * RPT, RPT)])
    plsc.subcore_barrier()

    @pl.when(wid < EXTRA)
    def _():
        row = NW * CPW + wid
        pltpu.sync_copy(ei3.at[0, row], sidx.at[0])
        pltpu.sync_copy(ei3.at[1, row], didx.at[0])
        pltpu.async_copy(u_hbm.at[sidx.at[0]], rows.at[0], gsem).wait()
        pltpu.async_copy(rows.at[0], acc_sh.at[didx.at[0]], ssem,
                         add=True).wait()

    @pl.loop(0, SUPER)
    def _(g):
        row0 = wid * CPW + g * K
        pltpu.sync_copy(ei3.at[0, pl.ds(row0, K)], sidx)
        pltpu.sync_copy(ei3.at[1, pl.ds(row0, K)], didx)
        gds = [pltpu.async_copy(u_hbm.at[sidx.at[j]], rows.at[j], gsem)
               for j in range(K)]
        sds = []
        for j in range(K):
            gds[j].wait()
            sds.append(
                pltpu.async_copy(rows.at[j], acc_sh.at[didx.at[j]], ssem,
                                 add=True))
        for dsc in sds:
            dsc.wait()

    plsc.subcore_barrier()

    @pl.when(c == 0)
    def _():
        pltpu.sync_copy(acc_sh.at[pl.ds(s * RPT, RPT)],
                        outa.at[pl.ds(s * RPT, RPT)])

    @pl.when(c == 1)
    def _():
        pltpu.sync_copy(acc_sh.at[pl.ds(s * RPT, RPT)],
                        outb.at[pl.ds(s * RPT, RPT)])


@functools.partial(
    pl.kernel,
    out_type=[_P, _P],
    mesh=_mesh,
    scratch_types=[
        pltpu.VMEM((B0, CH), jnp.float32),
        pltpu.VMEM((B0, CH), jnp.float32),
        pltpu.VMEM((B0, 3), jnp.float32),
        pltpu.VMEM((B0, CH), jnp.float32),
        pltpu.VMEM((B0, CH), jnp.float32),
    ],
    compiler_params=_cmp_params,
)
def _sc_prep(dp0, dp1, x_hbm, u1t, d8t, pa, pb, px, pu, pd):
    c = lax.axis_index("c")
    s = lax.axis_index("s")
    wid = s * NC + c

    def block(off, size, xsize):
        base = wid * DR + off
        pltpu.sync_copy(dp0.at[pl.ds(base, size)], pa.at[pl.ds(0, size)])
        pltpu.sync_copy(dp1.at[pl.ds(base, size)], pb.at[pl.ds(0, size)])
        pltpu.sync_copy(x_hbm.at[pl.ds(base, xsize)], px.at[pl.ds(0, xsize)])

        @pl.loop(0, size // 2)
        def _(i):
            rowv, colv = _flatpos(i)
            a = plsc.load_gather(pa, [rowv, colv])
            b = plsc.load_gather(pb, [rowv, colv])
            dv = _newton_rsqrt(a + b + 1.0)
            xg = plsc.load_gather(px, [rowv, jnp.minimum(colv, 2)])
            plsc.store_scatter(pd, [rowv, colv], dv)
            plsc.store_scatter(pu, [rowv, colv], dv * xg)

        pltpu.sync_copy(pu.at[pl.ds(0, size)], u1t.at[pl.ds(base, size)])
        pltpu.sync_copy(pd.at[pl.ds(0, size)], d8t.at[pl.ds(base, size)])

    block(0, B0, B0)

    @pl.when(wid == NW - 1)
    def _():
        block(B0, B1A, B1B)

    @pl.when(wid < NW - 1)
    def _():
        block(B0, B1A, B1A)


DS = 1564


@functools.partial(
    pl.kernel,
    out_type=_P,
    mesh=_mesh,
    scratch_types=[
        pltpu.VMEM((DS, CH), jnp.float32),
        pltpu.VMEM((DS, CH), jnp.float32),
        pltpu.VMEM((DS, CH), jnp.float32),
        pltpu.VMEM((DS, CH), jnp.float32),
        pltpu.VMEM((DS, CH), jnp.float32),
        pltpu.VMEM((DS * CH,), jnp.float32),
        pltpu.VMEM((DS * CH,), jnp.float32),
        pltpu.VMEM((64,), jnp.float32),
        pltpu.VMEM((CH, 32), jnp.float32),
        pltpu.VMEM((32,), jnp.float32),
        pltpu.VMEM((3, 32), jnp.float32),
        pltpu.SMEM((CH, 32), jnp.float32),
        pltpu.SMEM((32,), jnp.float32),
        pltpu.SMEM((3, 32), jnp.float32),
    ],
    compiler_params=_cmp_params,
)
def _sc_dense(d8t, s1a, s1b, u1t, w1_hbm, b1_hbm, w2t_hbm, u2t,
              qd, qs, qt, qu, qo, ybuf, dbuf, zbuf, w1v, b1vv, w2v,
              w1, b1v, w2):
    c = lax.axis_index("c")
    s = lax.axis_index("s")
    wid = s * NC + c
    pltpu.sync_copy(w1_hbm, w1v)
    pltpu.sync_copy(b1_hbm, b1vv)
    pltpu.sync_copy(w2t_hbm, w2v)
    for r in range(CH):
        for cb in (0, 16):
            vv = w1v[r, pl.ds(cb, 16)]
            for j in range(16):
                w1[r, cb + j] = vv[j]
    for cb in (0, 16):
        vv = b1vv[pl.ds(cb, 16)]
        for j in range(16):
            b1v[cb + j] = vv[j]
    for r in range(3):
        for cb in (0, 16):
            vv = w2v[r, pl.ds(cb, 16)]
            for j in range(16):
                w2[r, cb + j] = vv[j]
    iota = lax.iota(jnp.int32, 16)
    zero16 = jnp.zeros((16,), jnp.float32)
    for t in range(4):
        zbuf[pl.ds(t * 16, 16)] = zero16
    colv8 = iota & 7
    rbase = lax.shift_right_logical(iota, 3)
    basepat = jnp.where(colv8 < 3, colv8 * 16 + rbase, 48)

    for off in (0, DS):
        base = wid * DR + off
        pltpu.sync_copy(d8t.at[pl.ds(base, DS)], qd)
        pltpu.sync_copy(s1a.at[pl.ds(base, DS)], qs)
        pltpu.sync_copy(s1b.at[pl.ds(base, DS)], qt)
        pltpu.sync_copy(u1t.at[pl.ds(base, DS)], qu)

        @pl.loop(0, DS // 2)
        def _(i):
            rowv, colv = _flatpos(i)
            dv = plsc.load_gather(qd, [rowv, colv])
            sv = (plsc.load_gather(qs, [rowv, colv])
                  + plsc.load_gather(qt, [rowv, colv])
                  + plsc.load_gather(qu, [rowv, colv]))
            dbuf[pl.ds(i * 16, 16)] = dv
            ybuf[pl.ds(i * 16, 16)] = dv * sv

        NGROUPS = DS // 32 + 1

        @pl.loop(0, NGROUPS)
        def _(g):
            ng = jnp.minimum(g * 32, DS - 32)
            fb = ng * CH
            yca = [plsc.load_gather(ybuf, [iota * CH + (fb + cc)])
                   for cc in range(CH)]
            ycb = [plsc.load_gather(ybuf, [iota * CH + (fb + 128 + cc)])
                   for cc in range(CH)]
            za = [zero16, zero16, zero16]
            zb = [zero16, zero16, zero16]
            for k in range(32):
                wk = [w1[cc, k] for cc in range(CH)]
                ha = yca[0] * wk[0]
                hb = ycb[0] * wk[0]
                for cc in range(1, CH):
                    ha = ha + yca[cc] * wk[cc]
                    hb = hb + ycb[cc] * wk[cc]
                bk = b1v[k]
                ha = jnp.maximum(ha + bk, 0.0)
                hb = jnp.maximum(hb + bk, 0.0)
                for r in range(3):
                    wr = w2[r, k]
                    za[r] = za[r] + ha * wr
                    zb[r] = zb[r] + hb * wr
            for half, zz in ((0, za), (1, zb)):
                zbuf[pl.ds(0, 16)] = zz[0]
                zbuf[pl.ds(16, 16)] = zz[1]
                zbuf[pl.ds(32, 16)] = zz[2]
                for v in range(CH):
                    zg = plsc.load_gather(zbuf, [basepat + 2 * v])
                    dv = dbuf[pl.ds(fb + half * 128 + v * 16, 16)]
                    rowv = rbase + (ng + half * 16 + 2 * v)
                    plsc.store_scatter(qo, [rowv, colv8], dv * zg)

        pltpu.sync_copy(qo, u2t.at[pl.ds(base, DS)])


@functools.partial(
    pl.kernel,
    out_type=jax.ShapeDtypeStruct((N, 3), jnp.float32),
    mesh=_mesh,
    scratch_types=[
        pltpu.VMEM((B0, CH), jnp.float32),
        pltpu.VMEM((B0, CH), jnp.float32),
        pltpu.VMEM((B0, CH), jnp.float32),
        pltpu.VMEM((B0, CH), jnp.float32),
        pltpu.VMEM((B0 * CH,), jnp.float32),
        pltpu.VMEM((B0, 3), jnp.float32),
        pltpu.VMEM((16,), jnp.float32),
    ],
    compiler_params=_cmp_params,
)
def _sc_final(d8t, s2a, s2b, u2t, b2_hbm, dxout,
              rd, rs, rt, ru, vbuf, dxbuf, b2b):
    c = lax.axis_index("c")
    s = lax.axis_index("s")
    wid = s * NC + c
    pltpu.sync_copy(b2_hbm, b2b)
    iota = lax.iota(jnp.int32, 16)
    colv8 = iota & 7
    b2vec = plsc.load_gather(b2b, [jnp.where(colv8 < 3, colv8, 3)])
    pats, dxrow, dxcol = [], [], []
    for r in range(3):
        f = iota + 16 * r
        n3 = f // 3
        c3 = f - 3 * n3
        pats.append(CH * n3 + c3)
        dxrow.append(n3)
        dxcol.append(c3)

    def block(off, size):
        base = wid * DR + off
        pltpu.sync_copy(d8t.at[pl.ds(base, size)], rd.at[pl.ds(0, size)])
        pltpu.sync_copy(s2a.at[pl.ds(base, size)], rs.at[pl.ds(0, size)])
        pltpu.sync_copy(s2b.at[pl.ds(base, size)], rt.at[pl.ds(0, size)])
        pltpu.sync_copy(u2t.at[pl.ds(base, size)], ru.at[pl.ds(0, size)])

        @pl.loop(0, size // 2)
        def _(i):
            rowv, colv = _flatpos(i)
            dv = plsc.load_gather(rd, [rowv, colv])
            sv = (plsc.load_gather(rs, [rowv, colv])
                  + plsc.load_gather(rt, [rowv, colv])
                  + plsc.load_gather(ru, [rowv, colv]))
            vbuf[pl.ds(i * 16, 16)] = dv * sv + b2vec

        @pl.loop(0, size // 16 + 1)
        def _(q):
            mq = jnp.minimum(q * 16, size - 16)
            for r in range(3):
                dxv = plsc.load_gather(vbuf, [pats[r] + CH * mq])
                plsc.store_scatter(dxbuf, [dxrow[r] + mq, dxcol[r]], dxv)

        pltpu.sync_copy(dxbuf.at[pl.ds(0, size)],
                        dxout.at[pl.ds(base, size)])

    block(0, B0)

    @pl.when(wid == NW - 1)
    def _():
        block(B0, B1B)

    @pl.when(wid < NW - 1)
    def _():
        block(B0, B1A)


def kernel(x, edge_index, W1, b1, W2, b2):
    ei3 = edge_index.astype(jnp.int32).reshape(2, NCHUNKS, CHUNK)
    W1p = jnp.pad(W1, ((0, CH - 3), (0, 0)))
    W2T = W2.T
    b2p = jnp.pad(b2, (0, 13))
    zeros8 = jnp.zeros((NP, CH), jnp.float32)
    ones8 = jnp.ones((CHUNK, CH), jnp.float32)

    dp0, dp1 = _sc_deg(ei3, zeros8, ones8)
    u1t, d8t = _sc_prep(dp0, dp1, x)
    s1a, s1b = _sc_agg(ei3, u1t, zeros8)
    u2t = _sc_dense(d8t, s1a, s1b, u1t, W1p, b1, W2T)
    s2a, s2b = _sc_agg(ei3, u2t, zeros8)
    return _sc_final(d8t, s2a, s2b, u2t, b2p)


# speedup vs baseline: 76.5316x; 1.0304x over previous
import functools

import jax
import jax.numpy as jnp
from jax import lax
from jax.experimental import pallas as pl
from jax.experimental.pallas import tpu as pltpu
from jax.experimental.pallas import tpu_sc as plsc

N = 100000
NP = 100096
CH = 8
E = 1600000
NC = 2
NS = 16
NW = NC * NS
CHUNK = 128
NCHUNKS = E // CHUNK
CPW = NCHUNKS // NW
EXTRA = NCHUNKS - CPW * NW
K = 39
SUPER = CPW // K
RPT = NP // NS
DR = NP // NW
B0, B1A, B1B = 1568, 1560, 1464

_mesh = plsc.VectorSubcoreMesh(core_axis_name="c", subcore_axis_name="s")
_agg_params = pltpu.CompilerParams(use_tc_tiling_on_sc=False)
_cmp_params = pltpu.CompilerParams(use_tc_tiling_on_sc=False,
                                   needs_layout_passes=False)
_P = jax.ShapeDtypeStruct((NP, CH), jnp.float32)


def _newton_rsqrt(v):
    bi = plsc.bitcast(v, jnp.int32)
    y = plsc.bitcast(jnp.int32(0x5F3759DF) - lax.shift_right_logical(bi, 1),
                     jnp.float32)
    vh = 0.5 * v
    for _ in range(3):
        y = y * (1.5 - vh * y * y)
    return y


def _flatpos(i):
    iota = lax.iota(jnp.int32, 16)
    return lax.shift_right_logical(iota, 3) + 2 * i, iota & 7


@functools.partial(
    pl.kernel,
    out_type=[_P, _P],
    mesh=_mesh,
    scratch_types=[
        pltpu.VMEM((K, CHUNK), jnp.int32),
        pltpu.VMEM((CHUNK, CH), jnp.float32),
        pltpu.VMEM_SHARED((NP, CH), jnp.float32),
        pltpu.SemaphoreType.DMA,
    ],
    compiler_params=_agg_params,
)
def _sc_deg(ei3, zeros_hbm, ones_hbm, out0, out1, didx, ones_v, acc_sh, ssem):
    c = lax.axis_index("c")
    s = lax.axis_index("s")
    wid = s * NC + c
    pltpu.sync_copy(zeros_hbm, acc_sh.at[pl.ds(s * RPT, RPT)])
    pltpu.sync_copy(ones_hbm, ones_v)
    plsc.subcore_barrier()

    @pl.when(wid < EXTRA)
    def _():
        row = NW * CPW + wid
        pltpu.sync_copy(ei3.at[1, row], didx.at[0])
        pltpu.async_copy(ones_v, acc_sh.at[didx.at[0]], ssem, add=True).wait()

    @pl.loop(0, SUPER)
    def _(g):
        row0 = wid * CPW + g * K
        pltpu.sync_copy(ei3.at[1, pl.ds(row0, K)], didx)
        descs = [
            pltpu.async_copy(ones_v, acc_sh.at[didx.at[j]], ssem, add=True)
            for j in range(K)
        ]
        for dsc in descs:
            dsc.wait()

    plsc.subcore_barrier()

    @pl.when(c == 0)
    def _():
        pltpu.sync_copy(acc_sh.at[pl.ds(s * RPT, RPT)],
                        out0.at[pl.ds(s * RPT, RPT)])

    @pl.when(c == 1)
    def _():
        pltpu.sync_copy(acc_sh.at[pl.ds(s * RPT, RPT)],
                        out1.at[pl.ds(s * RPT, RPT)])


@functools.partial(
    pl.kernel,
    out_type=[_P, _P],
    mesh=_mesh,
    scratch_types=[
        pltpu.VMEM((K, CHUNK), jnp.int32),
        pltpu.VMEM((K, CHUNK), jnp.int32),
        pltpu.VMEM((K, CHUNK, CH), jnp.float32),
        pltpu.VMEM_SHARED((NP, CH), jnp.float32),
        pltpu.SemaphoreType.DMA,
        pltpu.SemaphoreType.DMA,
    ],
    compiler_params=_agg_params,
)
def _sc_agg(ei3, u_hbm, zeros_hbm, outa, outb,
            sidx, didx, rows, acc_sh, gsem, ssem):
    c = lax.axis_index("c")
    s = lax.axis_index("s")
    wid = s * NC + c
    pltpu.sync_copy(zeros_hbm, acc_sh.at[pl.ds(s * RPT, RPT)])
    plsc.subcore_barrier()

    @pl.when(wid < EXTRA)
    def _():
        row = NW * CPW + wid
        pltpu.sync_copy(ei3.at[0, row], sidx.at[0])
        pltpu.sync_copy(ei3.at[1, row], didx.at[0])
        pltpu.async_copy(u_hbm.at[sidx.at[0]], rows.at[0], gsem).wait()
        pltpu.async_copy(rows.at[0], acc_sh.at[didx.at[0]], ssem,
                         add=True).wait()

    @pl.loop(0, SUPER)
    def _(g):
        row0 = wid * CPW + g * K
        pltpu.sync_copy(ei3.at[0, pl.ds(row0, K)], sidx)
        pltpu.sync_copy(ei3.at[1, pl.ds(row0, K)], didx)
        gds = [pltpu.async_copy(u_hbm.at[sidx.at[j]], rows.at[j], gsem)
               for j in range(K)]
        sds = []
        for j in range(K):
            gds[j].wait()
            sds.append(
                pltpu.async_copy(rows.at[j], acc_sh.at[didx.at[j]], ssem,
                                 add=True))
        for dsc in sds:
            dsc.wait()

    plsc.subcore_barrier()

    @pl.when(c == 0)
    def _():
        pltpu.sync_copy(acc_sh.at[pl.ds(s * RPT, RPT)],
                        outa.at[pl.ds(s * RPT, RPT)])

    @pl.when(c == 1)
    def _():
        pltpu.sync_copy(acc_sh.at[pl.ds(s * RPT, RPT)],
                        outb.at[pl.ds(s * RPT, RPT)])


@functools.partial(
    pl.kernel,
    out_type=[_P, _P],
    mesh=_mesh,
    scratch_types=[
        pltpu.VMEM((B0, CH), jnp.float32),
        pltpu.VMEM((B0, CH), jnp.float32),
        pltpu.VMEM((B0, 3), jnp.float32),
        pltpu.VMEM((B0, CH), jnp.float32),
        pltpu.VMEM((B0, CH), jnp.float32),
    ],
    compiler_params=_cmp_params,
)
def _sc_prep(dp0, dp1, x_hbm, u1t, d8t, pa, pb, px, pu, pd):
    c = lax.axis_index("c")
    s = lax.axis_index("s")
    wid = s * NC + c

    def block(off, size, xsize):
        base = wid * DR + off
        pltpu.sync_copy(dp0.at[pl.ds(base, size)], pa.at[pl.ds(0, size)])
        pltpu.sync_copy(dp1.at[pl.ds(base, size)], pb.at[pl.ds(0, size)])
        pltpu.sync_copy(x_hbm.at[pl.ds(base, xsize)], px.at[pl.ds(0, xsize)])

        @pl.loop(0, size // 2)
        def _(i):
            rowv, colv = _flatpos(i)
            a = plsc.load_gather(pa, [rowv, colv])
            b = plsc.load_gather(pb, [rowv, colv])
            dv = _newton_rsqrt(a + b + 1.0)
            xg = plsc.load_gather(px, [rowv, jnp.minimum(colv, 2)])
            plsc.store_scatter(pd, [rowv, colv], dv)
            plsc.store_scatter(pu, [rowv, colv], dv * xg)

        pltpu.sync_copy(pu.at[pl.ds(0, size)], u1t.at[pl.ds(base, size)])
        pltpu.sync_copy(pd.at[pl.ds(0, size)], d8t.at[pl.ds(base, size)])

    block(0, B0, B0)

    @pl.when(wid == NW - 1)
    def _():
        block(B0, B1A, B1B)

    @pl.when(wid < NW - 1)
    def _():
        block(B0, B1A, B1A)


DS = 1564


@functools.partial(
    pl.kernel,
    out_type=_P,
    mesh=_mesh,
    scratch_types=[
        pltpu.VMEM((DS, CH), jnp.float32),
        pltpu.VMEM((DS, CH), jnp.float32),
        pltpu.VMEM((DS, CH), jnp.float32),
        pltpu.VMEM((DS, CH), jnp.float32),
        pltpu.VMEM((DS, CH), jnp.float32),
        pltpu.VMEM((DS * CH,), jnp.float32),
        pltpu.VMEM((DS * CH,), jnp.float32),
        pltpu.VMEM((64,), jnp.float32),
        pltpu.VMEM((CH, 32), jnp.float32),
        pltpu.VMEM((32,), jnp.float32),
        pltpu.VMEM((3, 32), jnp.float32),
        pltpu.SMEM((CH, 32), jnp.float32),
        pltpu.SMEM((32,), jnp.float32),
        pltpu.SMEM((3, 32), jnp.float32),
    ],
    compiler_params=_cmp_params,
)
def _sc_dense(d8t, s1a, s1b, u1t, w1_hbm, b1_hbm, w2t_hbm, u2t,
              qd, qs, qt, qu, qo, ybuf, dbuf, zbuf, w1v, b1vv, w2v,
              w1, b1v, w2):
    c = lax.axis_index("c")
    s = lax.axis_index("s")
    wid = s * NC + c
    pltpu.sync_copy(w1_hbm, w1v)
    pltpu.sync_copy(b1_hbm, b1vv)
    pltpu.sync_copy(w2t_hbm, w2v)
    for r in range(CH):
        for cb in (0, 16):
            vv = w1v[r, pl.ds(cb, 16)]
            for j in range(16):
                w1[r, cb + j] = vv[j]
    for cb in (0, 16):
        vv = b1vv[pl.ds(cb, 16)]
        for j in range(16):
            b1v[cb + j] = vv[j]
    for r in range(3):
        for cb in (0, 16):
            vv = w2v[r, pl.ds(cb, 16)]
            for j in range(16):
                w2[r, cb + j] = vv[j]
    iota = lax.iota(jnp.int32, 16)
    zero16 = jnp.zeros((16,), jnp.float32)
    for t in range(4):
        zbuf[pl.ds(t * 16, 16)] = zero16
    colv8 = iota & 7
    rbase = lax.shift_right_logical(iota, 3)
    basepat = jnp.where(colv8 < 3, colv8 * 16 + rbase, 48)

    for off in (0, DS):
        base = wid * DR + off
        pltpu.sync_copy(d8t.at[pl.ds(base, DS)], qd)
        pltpu.sync_copy(s1a.at[pl.ds(base, DS)], qs)
        pltpu.sync_copy(s1b.at[pl.ds(base, DS)], qt)
        pltpu.sync_copy(u1t.at[pl.ds(base, DS)], qu)

        @pl.loop(0, DS // 2)
        def _(i):
            rowv, colv = _flatpos(i)
            dv = plsc.load_gather(qd, [rowv, colv])
            sv = (plsc.load_gather(qs, [rowv, colv])
                  + plsc.load_gather(qt, [rowv, colv])
                  + plsc.load_gather(qu, [rowv, colv]))
            dbuf[pl.ds(i * 16, 16)] = dv
            ybuf[pl.ds(i * 16, 16)] = dv * sv

        NGROUPS = DS // 32 + 1

        @pl.loop(0, NGROUPS)
        def _(g):
            ng = jnp.minimum(g * 32, DS - 32)
            fb = ng * CH
            yca = [plsc.load_gather(ybuf, [iota * CH + (fb + cc)])
                   for cc in range(CH)]
            ycb = [plsc.load_gather(ybuf, [iota * CH + (fb + 128 + cc)])
                   for cc in range(CH)]
            za = [zero16, zero16, zero16]
            zb = [zero16, zero16, zero16]
            for k in range(32):
                wk = [w1[cc, k] for cc in range(CH)]
                ha = yca[0] * wk[0]
                hb = ycb[0] * wk[0]
                for cc in range(1, CH):
                    ha = ha + yca[cc] * wk[cc]
                    hb = hb + ycb[cc] * wk[cc]
                bk = b1v[k]
                ha = jnp.maximum(ha + bk, 0.0)
                hb = jnp.maximum(hb + bk, 0.0)
                for r in range(3):
                    wr = w2[r, k]
                    za[r] = za[r] + ha * wr
                    zb[r] = zb[r] + hb * wr
            for half, zz in ((0, za), (1, zb)):
                zbuf[pl.ds(0, 16)] = zz[0]
                zbuf[pl.ds(16, 16)] = zz[1]
                zbuf[pl.ds(32, 16)] = zz[2]
                for v in range(CH):
                    zg = plsc.load_gather(zbuf, [basepat + 2 * v])
                    dv = dbuf[pl.ds(fb + half * 128 + v * 16, 16)]
                    rowv = rbase + (ng + half * 16 + 2 * v)
                    plsc.store_scatter(qo, [rowv, colv8], dv * zg)

        pltpu.sync_copy(qo, u2t.at[pl.ds(base, DS)])


@functools.partial(
    pl.kernel,
    out_type=jax.ShapeDtypeStruct((N, 3), jnp.float32),
    mesh=_mesh,
    scratch_types=[
        pltpu.VMEM((B0, CH), jnp.float32),
        pltpu.VMEM((B0, CH), jnp.float32),
        pltpu.VMEM((B0, CH), jnp.float32),
        pltpu.VMEM((B0, CH), jnp.float32),
        pltpu.VMEM((B0 * CH,), jnp.float32),
        pltpu.VMEM((B0, 3), jnp.float32),
        pltpu.VMEM((16,), jnp.float32),
    ],
    compiler_params=_cmp_params,
)
def _sc_final(d8t, s2a, s2b, u2t, b2_hbm, dxout,
              rd, rs, rt, ru, vbuf, dxbuf, b2b):
    c = lax.axis_index("c")
    s = lax.axis_index("s")
    wid = s * NC + c
    pltpu.sync_copy(b2_hbm, b2b)
    iota = lax.iota(jnp.int32, 16)
    colv8 = iota & 7
    b2vec = plsc.load_gather(b2b, [jnp.where(colv8 < 3, colv8, 3)])
    pats, dxrow, dxcol = [], [], []
    for r in range(3):
        f = iota + 16 * r
        n3 = f // 3
        c3 = f - 3 * n3
        pats.append(CH * n3 + c3)
        dxrow.append(n3)
        dxcol.append(c3)

    def block(off, size):
        base = wid * DR + off
        pltpu.sync_copy(d8t.at[pl.ds(base, size)], rd.at[pl.ds(0, size)])
        pltpu.sync_copy(s2a.at[pl.ds(base, size)], rs.at[pl.ds(0, size)])
        pltpu.sync_copy(s2b.at[pl.ds(base, size)], rt.at[pl.ds(0, size)])
        pltpu.sync_copy(u2t.at[pl.ds(base, size)], ru.at[pl.ds(0, size)])

        @pl.loop(0, size // 2)
        def _(i):
            rowv, colv = _flatpos(i)
            dv = plsc.load_gather(rd, [rowv, colv])
            sv = (plsc.load_gather(rs, [rowv, colv])
                  + plsc.load_gather(rt, [rowv, colv])
                  + plsc.load_gather(ru, [rowv, colv]))
            vbuf[pl.ds(i * 16, 16)] = dv * sv + b2vec

        @pl.loop(0, size // 16 + 1)
        def _(q):
            mq = jnp.minimum(q * 16, size - 16)
            for r in range(3):
                dxv = plsc.load_gather(vbuf, [pats[r] + CH * mq])
                plsc.store_scatter(dxbuf, [dxrow[r] + mq, dxcol[r]], dxv)

        pltpu.sync_copy(dxbuf.at[pl.ds(0, size)],
                        dxout.at[pl.ds(base, size)])

    block(0, B0)

    @pl.when(wid == NW - 1)
    def _():
        block(B0, B1B)

    @pl.when(wid < NW - 1)
    def _():
        block(B0, B1A)


def kernel(x, edge_index, W1, b1, W2, b2):
    ei3 = edge_index.astype(jnp.int32).reshape(2, NCHUNKS, CHUNK)
    W1p = jnp.pad(W1, ((0, CH - 3), (0, 0)))
    W2T = W2.T
    b2p = jnp.pad(b2, (0, 13))
    zeros8 = jnp.zeros((RPT, CH), jnp.float32)
    ones8 = jnp.ones((CHUNK, CH), jnp.float32)

    dp0, dp1 = _sc_deg(ei3, zeros8, ones8)
    u1t, d8t = _sc_prep(dp0, dp1, x)
    s1a, s1b = _sc_agg(ei3, u1t, zeros8)
    u2t = _sc_dense(d8t, s1a, s1b, u1t, W1p, b1, W2T)
    s2a, s2b = _sc_agg(ei3, u2t, zeros8)
    return _sc_final(d8t, s2a, s2b, u2t, b2p)
